# baseline (speedup 1.0000x reference)
import functools

import jax
import jax.numpy as jnp
from jax import lax
from jax.experimental import pallas as pl
from jax.experimental.pallas import tpu as pltpu
from jax.experimental.pallas import tpu_sc as plsc

N = 10000
NP = 10240
E = 320000
D = 128
F = 64
G = 64
GP = 128
NW = 32
EPT = E // NW
CH = 2000
ACH = 4000
NCHA = E // ACH
L = 16
TB = 512

_mesh = plsc.VectorSubcoreMesh(core_axis_name="c", subcore_axis_name="s")
_sc_params = pltpu.CompilerParams(needs_layout_passes=False)
_f32 = jnp.float32
_i32 = jnp.int32


def _wid():
    return lax.axis_index("c") * 16 + lax.axis_index("s")


def _deg_body(col_hbm, ew_hbm, parts_hbm, colb, ewb, degp):
    wid = _wid()

    def zero(i, _):
        degp[pl.ds(i * L, L)] = jnp.zeros((L,), _f32)
        return _

    lax.fori_loop(0, NP // L, zero, None)
    base = pl.multiple_of(wid * EPT, 8)

    def chunk(cc, _):
        off = pl.multiple_of(base + cc * CH, 8)
        pltpu.sync_copy(col_hbm.at[pl.ds(off, CH)], colb)
        pltpu.sync_copy(ew_hbm.at[pl.ds(off, CH)], ewb)

        @plsc.parallel_loop(0, CH // L, 1, unroll=5)
        def _(g):
            d = pl.ds(g * L, L)
            plsc.addupdate_scatter(degp, [colb[d]], ewb[d])

        return _

    lax.fori_loop(0, EPT // CH, chunk, None)
    pltpu.sync_copy(degp, parts_hbm.at[wid])


_deg_kernel = pl.kernel(
    _deg_body,
    out_type=jax.ShapeDtypeStruct((NW, NP), _f32),
    mesh=_mesh,
    compiler_params=_sc_params,
    scratch_types=[
        pltpu.VMEM((CH,), _i32),
        pltpu.VMEM((CH,), _f32),
        pltpu.VMEM((NP,), _f32),
    ],
)


def _norm_body(row_hbm, col_hbm, ew_hbm, dinv_hbm, norm_hbm, pk_hbm,
               dinvb, rowb, colb, ewb, normb, packb):
    wid = _wid()
    pltpu.sync_copy(dinv_hbm, dinvb)
    base = pl.multiple_of(wid * EPT, 8)

    def chunk(cc, _):
        off = pl.multiple_of(base + cc * CH, 8)
        pltpu.sync_copy(row_hbm.at[pl.ds(off, CH)], rowb)
        pltpu.sync_copy(col_hbm.at[pl.ds(off, CH)], colb)
        pltpu.sync_copy(ew_hbm.at[pl.ds(off, CH)], ewb)

        @plsc.parallel_loop(0, CH // L, 1, unroll=5)
        def _(g):
            d = pl.ds(g * L, L)
            rv = rowb[d]
            cv = colb[d]
            dr = plsc.load_gather(dinvb, [rv])
            dc = plsc.load_gather(dinvb, [cv])
            normb[d] = dr * ewb[d] * dc
            packb[d] = jnp.bitwise_or(lax.shift_left(rv, 16), cv)

        pltpu.sync_copy(normb, norm_hbm.at[pl.ds(off, CH)])
        pltpu.sync_copy(packb, pk_hbm.at[pl.ds(off, CH)])
        return _

    lax.fori_loop(0, EPT // CH, chunk, None)


_norm_kernel = pl.kernel(
    _norm_body,
    out_type=(
        jax.ShapeDtypeStruct((E,), _f32),
        jax.ShapeDtypeStruct((E,), _i32),
    ),
    mesh=_mesh,
    compiler_params=_sc_params,
    scratch_types=[
        pltpu.VMEM((NP,), _f32),
        pltpu.VMEM((CH,), _i32),
        pltpu.VMEM((CH,), _i32),
        pltpu.VMEM((CH,), _f32),
        pltpu.VMEM((CH,), _f32),
        pltpu.VMEM((CH,), _i32),
    ],
)


EH = E // 2
NCHH = EH // ACH


def _agg_body(hlin_hbm, hself_hbm, pk_hbm, norm_hbm, b_hbm,
              hout_hbm,
              hc0, hc1, hc2, hc3, ag0, ag1, ag2, ag3, tmp,
              pkb0, normb0, pkb1, normb1, sem0, sem1, biasb, shared):
    c = lax.axis_index("c")
    s = lax.axis_index("s")
    u = lax.div(s, 2)
    odd = lax.rem(s, 2)
    j0 = (c * 8 + u) * 4

    for k, hc in enumerate((hc0, hc1, hc2, hc3)):
        pltpu.sync_copy(hlin_hbm.at[j0 + k], hc)
    pltpu.sync_copy(b_hbm, biasb)

    def zero_ref(ref):
        def z(i, _):
            ref[pl.ds(i * L, L)] = jnp.zeros((L,), _f32)
            return _
        lax.fori_loop(0, NP // L, z, None, unroll=4)

    @pl.when(odd == 0)
    def _():
        pltpu.sync_copy(hself_hbm.at[j0], ag0)
        pltpu.sync_copy(hself_hbm.at[j0 + 1], ag1)
        zero_ref(ag2)
        zero_ref(ag3)

    @pl.when(odd == 1)
    def _():
        pltpu.sync_copy(hself_hbm.at[j0 + 2], ag2)
        pltpu.sync_copy(hself_hbm.at[j0 + 3], ag3)
        zero_ref(ag0)
        zero_ref(ag1)

    ebase = odd * EH

    def issue(ci, pkb, normb, sem):
        off = pl.multiple_of(ebase + ci * ACH, 8)
        pltpu.async_copy(pk_hbm.at[pl.ds(off, ACH)], pkb, sem)
        pltpu.async_copy(norm_hbm.at[pl.ds(off, ACH)], normb, sem)

    def wait(pkb, normb, sem):
        pltpu.make_async_copy(pk_hbm.at[pl.ds(0, ACH)], pkb, sem).wait()
        pltpu.make_async_copy(norm_hbm.at[pl.ds(0, ACH)], normb, sem).wait()

    def compute(pkb, normb):
        @plsc.parallel_loop(0, ACH // L, 1, unroll=4)
        def _(g):
            d = pl.ds(g * L, L)
            pk = pkb[d]
            rv = lax.shift_right_logical(pk, 16)
            cv = jnp.bitwise_and(pk, 0xFFFF)
            nv = normb[d]
            plsc.addupdate_scatter(ag0, [cv], plsc.load_gather(hc0, [rv]) * nv)
            plsc.addupdate_scatter(ag1, [cv], plsc.load_gather(hc1, [rv]) * nv)
            plsc.addupdate_scatter(ag2, [cv], plsc.load_gather(hc2, [rv]) * nv)
            plsc.addupdate_scatter(ag3, [cv], plsc.load_gather(hc3, [rv]) * nv)

    issue(0, pkb0, normb0, sem0)

    def pair(i, _):
        issue(2 * i + 1, pkb1, normb1, sem1)
        wait(pkb0, normb0, sem0)
        compute(pkb0, normb0)

        @pl.when(i < NCHH // 2 - 1)
        def _():
            issue(2 * i + 2, pkb0, normb0, sem0)

        wait(pkb1, normb1, sem1)
        compute(pkb1, normb1)
        return _

    lax.fori_loop(0, NCHH // 2, pair, None)

    @pl.when(odd == 0)
    def _():
        pltpu.sync_copy(ag2, shared.at[s, 0])
        pltpu.sync_copy(ag3, shared.at[s, 1])

    @pl.when(odd == 1)
    def _():
        pltpu.sync_copy(ag0, shared.at[s, 0])
        pltpu.sync_copy(ag1, shared.at[s, 1])

    plsc.subcore_barrier()

    def add_from(srow, k, ref):
        pltpu.sync_copy(shared.at[srow, k], tmp)

        def a(i, _):
            d = pl.ds(i * L, L)
            ref[d] = ref[d] + tmp[d]
            return _

        lax.fori_loop(0, NP // L, a, None, unroll=4)

    def bias_splat(j):
        return plsc.load_gather(biasb, [jnp.full((L,), j, _i32)])

    def writeback(j, agg, outbuf):
        bs = bias_splat(j)

        def wbk(i, _):
            d = pl.ds(i * L, L)
            outbuf[d] = jnp.maximum(agg[d] + bs, 0.0)
            return _

        lax.fori_loop(0, NP // L, wbk, None, unroll=4)
        pltpu.sync_copy(outbuf, hout_hbm.at[j])

    @pl.when(odd == 0)
    def _():
        add_from(s + 1, 0, ag0)
        add_from(s + 1, 1, ag1)
        writeback(j0, ag0, hc0)
        writeback(j0 + 1, ag1, hc1)

    @pl.when(odd == 1)
    def _():
        add_from(s - 1, 0, ag2)
        add_from(s - 1, 1, ag3)
        writeback(j0 + 2, ag2, hc2)
        writeback(j0 + 3, ag3, hc3)


_agg_kernel = pl.kernel(
    _agg_body,
    out_type=jax.ShapeDtypeStruct((F, NP), _f32),
    mesh=_mesh,
    compiler_params=_sc_params,
    scratch_types=[
        pltpu.VMEM((NP,), _f32),
        pltpu.VMEM((NP,), _f32),
        pltpu.VMEM((NP,), _f32),
        pltpu.VMEM((NP,), _f32),
        pltpu.VMEM((NP,), _f32),
        pltpu.VMEM((NP,), _f32),
        pltpu.VMEM((NP,), _f32),
        pltpu.VMEM((NP,), _f32),
        pltpu.VMEM((NP,), _f32),
        pltpu.VMEM((ACH,), _i32),
        pltpu.VMEM((ACH,), _f32),
        pltpu.VMEM((ACH,), _i32),
        pltpu.VMEM((ACH,), _f32),
        pltpu.SemaphoreType.DMA,
        pltpu.SemaphoreType.DMA,
        pltpu.VMEM((F,), _f32),
        pltpu.VMEM_SHARED((16, 2, NP), _f32),
    ],
)


def _prep_body(x_ref, w_ref, dp_ref, h1t_ref, hself_ref, dinv_ref, invdeg_ref):
    deg = jnp.sum(dp_ref[...], axis=0, keepdims=True) + 1.0
    dinv = 1.0 / jnp.sqrt(deg)
    invdeg = dinv * dinv
    ht = lax.dot_general(w_ref[...].astype(jnp.bfloat16),
                         x_ref[...].astype(jnp.bfloat16),
                         (((0,), (1,)), ((), ())),
                         preferred_element_type=_f32)
    h1t_ref[...] = ht
    hself_ref[...] = ht * invdeg
    dinv_ref[...] = dinv
    invdeg_ref[...] = invdeg


def _mid_body(ht_ref, w_ref, invdeg_ref, hlt_ref, hself_ref):
    hlt = lax.dot_general(w_ref[...].astype(jnp.bfloat16),
                          ht_ref[...].astype(jnp.bfloat16),
                          (((0,), (0,)), ((), ())),
                          preferred_element_type=_f32)
    hlt_ref[...] = hlt
    hself_ref[...] = hlt * invdeg_ref[...]


def _final_body(h3t_ref, batch_ref, wl_ref, bl_ref, wl2_ref, bl2_ref,
                out_ref, acc_ref, cnt_ref):
    i = pl.program_id(0)
    bt = batch_ref[...]
    oh = (lax.broadcasted_iota(_i32, (G, TB), 0) == bt).astype(_f32)
    ps = lax.dot_general(h3t_ref[...], oh, (((1,), (1,)), ((), ())),
                         precision=lax.Precision.HIGHEST,
                         preferred_element_type=_f32)
    cs = lax.dot_general(jnp.ones((1, TB), _f32), oh, (((1,), (1,)), ((), ())),
                         precision=lax.Precision.HIGHEST,
                         preferred_element_type=_f32)

    @pl.when(i == 0)
    def _():
        acc_ref[...] = jnp.zeros_like(acc_ref)
        cnt_ref[...] = jnp.zeros_like(cnt_ref)

    acc_ref[...] += ps
    cnt_ref[...] += cs

    @pl.when(i == NP // TB - 1)
    def _():
        pooled = acc_ref[...] / jnp.maximum(cnt_ref[...], 1.0)
        t = lax.dot_general(pooled.astype(jnp.bfloat16),
                            wl_ref[...].astype(jnp.bfloat16),
                            (((0,), (0,)), ((), ())),
                            preferred_element_type=_f32)
        t = jnp.maximum(t + bl_ref[...], 0.0)
        o = lax.dot_general(t.astype(jnp.bfloat16),
                            wl2_ref[...].astype(jnp.bfloat16),
                            (((1,), (0,)), ((), ())),
                            preferred_element_type=_f32)
        out_ref[...] = o + bl2_ref[...]


def kernel(x, edge_index, edge_weight, batch,
           W1, b1, W2, b2, W3, b3, Wl, bl, Wl2, bl2):
    row = edge_index[0]
    col = edge_index[1]
    xP = jnp.zeros((NP, D), _f32).at[:N].set(x)
    batchP = jnp.concatenate([batch.astype(_i32), jnp.full((NP - N,), G, _i32)])

    deg_parts = _deg_kernel(col, edge_weight)

    nb = NP // TB
    h1t, hself1, dinv2d, invdeg2d = pl.pallas_call(
        _prep_body,
        grid=(nb,),
        in_specs=[
            pl.BlockSpec((TB, D), lambda i: (i, 0)),
            pl.BlockSpec((D, F), lambda i: (0, 0)),
            pl.BlockSpec((NW, TB), lambda i: (0, i)),
        ],
        out_specs=[
            pl.BlockSpec((F, TB), lambda i: (0, i)),
            pl.BlockSpec((F, TB), lambda i: (0, i)),
            pl.BlockSpec((1, TB), lambda i: (0, i)),
            pl.BlockSpec((1, TB), lambda i: (0, i)),
        ],
        out_shape=[
            jax.ShapeDtypeStruct((F, NP), _f32),
            jax.ShapeDtypeStruct((F, NP), _f32),
            jax.ShapeDtypeStruct((1, NP), _f32),
            jax.ShapeDtypeStruct((1, NP), _f32),
        ],
    )(xP, W1, deg_parts)

    norm, packed = _norm_kernel(row, col, edge_weight, jnp.reshape(dinv2d, (NP,)))

    def mid_matmul(ht, W):
        return pl.pallas_call(
            _mid_body,
            grid=(nb,),
            in_specs=[
                pl.BlockSpec((F, TB), lambda i: (0, i)),
                pl.BlockSpec((F, F), lambda i: (0, 0)),
                pl.BlockSpec((1, TB), lambda i: (0, i)),
            ],
            out_specs=[
                pl.BlockSpec((F, TB), lambda i: (0, i)),
                pl.BlockSpec((F, TB), lambda i: (0, i)),
            ],
            out_shape=[
                jax.ShapeDtypeStruct((F, NP), _f32),
                jax.ShapeDtypeStruct((F, NP), _f32),
            ],
        )(ht, W, invdeg2d)

    h2t = _agg_kernel(h1t, hself1, packed, norm, b1)
    hlt2, hself2 = mid_matmul(h2t, W2)
    h3t = _agg_kernel(hlt2, hself2, packed, norm, b2)
    hlt3, hself3 = mid_matmul(h3t, W3)
    h4t = _agg_kernel(hlt3, hself3, packed, norm, b3)

    out = pl.pallas_call(
        _final_body,
        grid=(nb,),
        in_specs=[
            pl.BlockSpec((F, TB), lambda i: (0, i)),
            pl.BlockSpec((1, TB), lambda i: (0, i)),
            pl.BlockSpec((F, 32), lambda i: (0, 0)),
            pl.BlockSpec((1, 32), lambda i: (0, 0)),
            pl.BlockSpec((32, 1), lambda i: (0, 0)),
            pl.BlockSpec((1, 1), lambda i: (0, 0)),
        ],
        out_specs=pl.BlockSpec((G, 1), lambda i: (0, 0)),
        out_shape=jax.ShapeDtypeStruct((G, 1), _f32),
        scratch_shapes=[
            pltpu.VMEM((F, G), _f32),
            pltpu.VMEM((1, G), _f32),
        ],
    )(h4t, jnp.reshape(batchP, (1, NP)), Wl,
      jnp.reshape(bl, (1, 32)), Wl2, jnp.reshape(bl2, (1, 1)))
    return out


# speedup vs baseline: 1.0021x; 1.0021x over previous
import functools

import jax
import jax.numpy as jnp
from jax import lax
from jax.experimental import pallas as pl
from jax.experimental.pallas import tpu as pltpu
from jax.experimental.pallas import tpu_sc as plsc

N = 10000
NP = 10240
E = 320000
D = 128
F = 64
G = 64
GP = 128
NW = 32
EPT = E // NW
CH = 2000
ACH = 4000
NCHA = E // ACH
L = 16
TB = 512

_mesh = plsc.VectorSubcoreMesh(core_axis_name="c", subcore_axis_name="s")
_sc_params = pltpu.CompilerParams(needs_layout_passes=False)
_f32 = jnp.float32
_i32 = jnp.int32


def _wid():
    return lax.axis_index("c") * 16 + lax.axis_index("s")


def _deg_body(col_hbm, ew_hbm, parts_hbm, colb, ewb, degp):
    wid = _wid()

    def zero(i, _):
        degp[pl.ds(i * L, L)] = jnp.zeros((L,), _f32)
        return _

    lax.fori_loop(0, NP // L, zero, None)
    base = pl.multiple_of(wid * EPT, 8)

    def chunk(cc, _):
        off = pl.multiple_of(base + cc * CH, 8)
        pltpu.sync_copy(col_hbm.at[pl.ds(off, CH)], colb)
        pltpu.sync_copy(ew_hbm.at[pl.ds(off, CH)], ewb)

        @plsc.parallel_loop(0, CH // L, 1, unroll=5)
        def _(g):
            d = pl.ds(g * L, L)
            plsc.addupdate_scatter(degp, [colb[d]], ewb[d])

        return _

    lax.fori_loop(0, EPT // CH, chunk, None)
    pltpu.sync_copy(degp, parts_hbm.at[wid])


_deg_kernel = pl.kernel(
    _deg_body,
    out_type=jax.ShapeDtypeStruct((NW, NP), _f32),
    mesh=_mesh,
    compiler_params=_sc_params,
    scratch_types=[
        pltpu.VMEM((CH,), _i32),
        pltpu.VMEM((CH,), _f32),
        pltpu.VMEM((NP,), _f32),
    ],
)


def _norm_body(row_hbm, col_hbm, ew_hbm, dinv_hbm, norm_hbm, pk_hbm,
               dinvb, rowb, colb, ewb, normb, packb):
    wid = _wid()
    pltpu.sync_copy(dinv_hbm, dinvb)
    base = pl.multiple_of(wid * EPT, 8)

    def chunk(cc, _):
        off = pl.multiple_of(base + cc * CH, 8)
        pltpu.sync_copy(row_hbm.at[pl.ds(off, CH)], rowb)
        pltpu.sync_copy(col_hbm.at[pl.ds(off, CH)], colb)
        pltpu.sync_copy(ew_hbm.at[pl.ds(off, CH)], ewb)

        @plsc.parallel_loop(0, CH // L, 1, unroll=5)
        def _(g):
            d = pl.ds(g * L, L)
            rv = rowb[d]
            cv = colb[d]
            dr = plsc.load_gather(dinvb, [rv])
            dc = plsc.load_gather(dinvb, [cv])
            normb[d] = dr * ewb[d] * dc
            packb[d] = jnp.bitwise_or(lax.shift_left(rv, 16), cv)

        pltpu.sync_copy(normb, norm_hbm.at[pl.ds(off, CH)])
        pltpu.sync_copy(packb, pk_hbm.at[pl.ds(off, CH)])
        return _

    lax.fori_loop(0, EPT // CH, chunk, None)


_norm_kernel = pl.kernel(
    _norm_body,
    out_type=(
        jax.ShapeDtypeStruct((E,), _f32),
        jax.ShapeDtypeStruct((E,), _i32),
    ),
    mesh=_mesh,
    compiler_params=_sc_params,
    scratch_types=[
        pltpu.VMEM((NP,), _f32),
        pltpu.VMEM((CH,), _i32),
        pltpu.VMEM((CH,), _i32),
        pltpu.VMEM((CH,), _f32),
        pltpu.VMEM((CH,), _f32),
        pltpu.VMEM((CH,), _i32),
    ],
)


EH = E // 2
NCHH = EH // ACH


def _agg_body(hlin_hbm, hself_hbm, pk_hbm, norm_hbm, b_hbm,
              hout_hbm,
              hc0, hc1, hc2, hc3, ag0, ag1, ag2, ag3, tmp,
              pkb0, normb0, pkb1, normb1, sem0, sem1, biasb, shared):
    c = lax.axis_index("c")
    s = lax.axis_index("s")
    u = lax.div(s, 2)
    odd = lax.rem(s, 2)
    j0 = (c * 8 + u) * 4

    for k, hc in enumerate((hc0, hc1, hc2, hc3)):
        pltpu.sync_copy(hlin_hbm.at[j0 + k], hc)
    pltpu.sync_copy(b_hbm, biasb)

    def zero_ref(ref):
        def z(i, _):
            ref[pl.ds(i * L, L)] = jnp.zeros((L,), _f32)
            return _
        lax.fori_loop(0, NP // L, z, None, unroll=4)

    @pl.when(odd == 0)
    def _():
        pltpu.sync_copy(hself_hbm.at[j0], ag0)
        pltpu.sync_copy(hself_hbm.at[j0 + 1], ag1)
        zero_ref(ag2)
        zero_ref(ag3)

    @pl.when(odd == 1)
    def _():
        pltpu.sync_copy(hself_hbm.at[j0 + 2], ag2)
        pltpu.sync_copy(hself_hbm.at[j0 + 3], ag3)
        zero_ref(ag0)
        zero_ref(ag1)

    ebase = odd * EH

    def issue(ci, pkb, normb, sem):
        off = pl.multiple_of(ebase + ci * ACH, 8)
        pltpu.async_copy(pk_hbm.at[pl.ds(off, ACH)], pkb, sem)
        pltpu.async_copy(norm_hbm.at[pl.ds(off, ACH)], normb, sem)

    def wait(pkb, normb, sem):
        pltpu.make_async_copy(pk_hbm.at[pl.ds(0, ACH)], pkb, sem).wait()
        pltpu.make_async_copy(norm_hbm.at[pl.ds(0, ACH)], normb, sem).wait()

    def compute(pkb, normb):
        @plsc.parallel_loop(0, ACH // L, 1, unroll=8)
        def _(g):
            d = pl.ds(g * L, L)
            pk = pkb[d]
            rv = lax.shift_right_logical(pk, 16)
            cv = jnp.bitwise_and(pk, 0xFFFF)
            nv = normb[d]
            plsc.addupdate_scatter(ag0, [cv], plsc.load_gather(hc0, [rv]) * nv)
            plsc.addupdate_scatter(ag1, [cv], plsc.load_gather(hc1, [rv]) * nv)
            plsc.addupdate_scatter(ag2, [cv], plsc.load_gather(hc2, [rv]) * nv)
            plsc.addupdate_scatter(ag3, [cv], plsc.load_gather(hc3, [rv]) * nv)

    issue(0, pkb0, normb0, sem0)

    def pair(i, _):
        issue(2 * i + 1, pkb1, normb1, sem1)
        wait(pkb0, normb0, sem0)
        compute(pkb0, normb0)

        @pl.when(i < NCHH // 2 - 1)
        def _():
            issue(2 * i + 2, pkb0, normb0, sem0)

        wait(pkb1, normb1, sem1)
        compute(pkb1, normb1)
        return _

    lax.fori_loop(0, NCHH // 2, pair, None)

    @pl.when(odd == 0)
    def _():
        pltpu.sync_copy(ag2, shared.at[s, 0])
        pltpu.sync_copy(ag3, shared.at[s, 1])

    @pl.when(odd == 1)
    def _():
        pltpu.sync_copy(ag0, shared.at[s, 0])
        pltpu.sync_copy(ag1, shared.at[s, 1])

    plsc.subcore_barrier()

    def add_from(srow, k, ref):
        pltpu.sync_copy(shared.at[srow, k], tmp)

        def a(i, _):
            d = pl.ds(i * L, L)
            ref[d] = ref[d] + tmp[d]
            return _

        lax.fori_loop(0, NP // L, a, None, unroll=4)

    def bias_splat(j):
        return plsc.load_gather(biasb, [jnp.full((L,), j, _i32)])

    def writeback(j, agg, outbuf):
        bs = bias_splat(j)

        def wbk(i, _):
            d = pl.ds(i * L, L)
            outbuf[d] = jnp.maximum(agg[d] + bs, 0.0)
            return _

        lax.fori_loop(0, NP // L, wbk, None, unroll=4)
        pltpu.sync_copy(outbuf, hout_hbm.at[j])

    @pl.when(odd == 0)
    def _():
        add_from(s + 1, 0, ag0)
        add_from(s + 1, 1, ag1)
        writeback(j0, ag0, hc0)
        writeback(j0 + 1, ag1, hc1)

    @pl.when(odd == 1)
    def _():
        add_from(s - 1, 0, ag2)
        add_from(s - 1, 1, ag3)
        writeback(j0 + 2, ag2, hc2)
        writeback(j0 + 3, ag3, hc3)


_agg_kernel = pl.kernel(
    _agg_body,
    out_type=jax.ShapeDtypeStruct((F, NP), _f32),
    mesh=_mesh,
    compiler_params=_sc_params,
    scratch_types=[
        pltpu.VMEM((NP,), _f32),
        pltpu.VMEM((NP,), _f32),
        pltpu.VMEM((NP,), _f32),
        pltpu.VMEM((NP,), _f32),
        pltpu.VMEM((NP,), _f32),
        pltpu.VMEM((NP,), _f32),
        pltpu.VMEM((NP,), _f32),
        pltpu.VMEM((NP,), _f32),
        pltpu.VMEM((NP,), _f32),
        pltpu.VMEM((ACH,), _i32),
        pltpu.VMEM((ACH,), _f32),
        pltpu.VMEM((ACH,), _i32),
        pltpu.VMEM((ACH,), _f32),
        pltpu.SemaphoreType.DMA,
        pltpu.SemaphoreType.DMA,
        pltpu.VMEM((F,), _f32),
        pltpu.VMEM_SHARED((16, 2, NP), _f32),
    ],
)


def _prep_body(x_ref, w_ref, dp_ref, h1t_ref, hself_ref, dinv_ref, invdeg_ref):
    deg = jnp.sum(dp_ref[...], axis=0, keepdims=True) + 1.0
    dinv = 1.0 / jnp.sqrt(deg)
    invdeg = dinv * dinv
    ht = lax.dot_general(w_ref[...].astype(jnp.bfloat16),
                         x_ref[...].astype(jnp.bfloat16),
                         (((0,), (1,)), ((), ())),
                         preferred_element_type=_f32)
    h1t_ref[...] = ht
    hself_ref[...] = ht * invdeg
    dinv_ref[...] = dinv
    invdeg_ref[...] = invdeg


def _mid_body(ht_ref, w_ref, invdeg_ref, hlt_ref, hself_ref):
    hlt = lax.dot_general(w_ref[...].astype(jnp.bfloat16),
                          ht_ref[...].astype(jnp.bfloat16),
                          (((0,), (0,)), ((), ())),
                          preferred_element_type=_f32)
    hlt_ref[...] = hlt
    hself_ref[...] = hlt * invdeg_ref[...]


def _final_body(h3t_ref, batch_ref, wl_ref, bl_ref, wl2_ref, bl2_ref,
                out_ref, acc_ref, cnt_ref):
    i = pl.program_id(0)
    bt = batch_ref[...]
    oh = (lax.broadcasted_iota(_i32, (G, TB), 0) == bt).astype(_f32)
    ps = lax.dot_general(h3t_ref[...], oh, (((1,), (1,)), ((), ())),
                         precision=lax.Precision.HIGHEST,
                         preferred_element_type=_f32)
    cs = lax.dot_general(jnp.ones((1, TB), _f32), oh, (((1,), (1,)), ((), ())),
                         precision=lax.Precision.HIGHEST,
                         preferred_element_type=_f32)

    @pl.when(i == 0)
    def _():
        acc_ref[...] = jnp.zeros_like(acc_ref)
        cnt_ref[...] = jnp.zeros_like(cnt_ref)

    acc_ref[...] += ps
    cnt_ref[...] += cs

    @pl.when(i == NP // TB - 1)
    def _():
        pooled = acc_ref[...] / jnp.maximum(cnt_ref[...], 1.0)
        t = lax.dot_general(pooled.astype(jnp.bfloat16),
                            wl_ref[...].astype(jnp.bfloat16),
                            (((0,), (0,)), ((), ())),
                            preferred_element_type=_f32)
        t = jnp.maximum(t + bl_ref[...], 0.0)
        o = lax.dot_general(t.astype(jnp.bfloat16),
                            wl2_ref[...].astype(jnp.bfloat16),
                            (((1,), (0,)), ((), ())),
                            preferred_element_type=_f32)
        out_ref[...] = o + bl2_ref[...]


def kernel(x, edge_index, edge_weight, batch,
           W1, b1, W2, b2, W3, b3, Wl, bl, Wl2, bl2):
    row = edge_index[0]
    col = edge_index[1]
    xP = jnp.zeros((NP, D), _f32).at[:N].set(x)
    batchP = jnp.concatenate([batch.astype(_i32), jnp.full((NP - N,), G, _i32)])

    deg_parts = _deg_kernel(col, edge_weight)

    nb = NP // TB
    h1t, hself1, dinv2d, invdeg2d = pl.pallas_call(
        _prep_body,
        grid=(nb,),
        in_specs=[
            pl.BlockSpec((TB, D), lambda i: (i, 0)),
            pl.BlockSpec((D, F), lambda i: (0, 0)),
            pl.BlockSpec((NW, TB), lambda i: (0, i)),
        ],
        out_specs=[
            pl.BlockSpec((F, TB), lambda i: (0, i)),
            pl.BlockSpec((F, TB), lambda i: (0, i)),
            pl.BlockSpec((1, TB), lambda i: (0, i)),
            pl.BlockSpec((1, TB), lambda i: (0, i)),
        ],
        out_shape=[
            jax.ShapeDtypeStruct((F, NP), _f32),
            jax.ShapeDtypeStruct((F, NP), _f32),
            jax.ShapeDtypeStruct((1, NP), _f32),
            jax.ShapeDtypeStruct((1, NP), _f32),
        ],
    )(xP, W1, deg_parts)

    norm, packed = _norm_kernel(row, col, edge_weight, jnp.reshape(dinv2d, (NP,)))

    def mid_matmul(ht, W):
        return pl.pallas_call(
            _mid_body,
            grid=(nb,),
            in_specs=[
                pl.BlockSpec((F, TB), lambda i: (0, i)),
                pl.BlockSpec((F, F), lambda i: (0, 0)),
                pl.BlockSpec((1, TB), lambda i: (0, i)),
            ],
            out_specs=[
                pl.BlockSpec((F, TB), lambda i: (0, i)),
                pl.BlockSpec((F, TB), lambda i: (0, i)),
            ],
            out_shape=[
                jax.ShapeDtypeStruct((F, NP), _f32),
                jax.ShapeDtypeStruct((F, NP), _f32),
            ],
        )(ht, W, invdeg2d)

    h2t = _agg_kernel(h1t, hself1, packed, norm, b1)
    hlt2, hself2 = mid_matmul(h2t, W2)
    h3t = _agg_kernel(hlt2, hself2, packed, norm, b2)
    hlt3, hself3 = mid_matmul(h3t, W3)
    h4t = _agg_kernel(hlt3, hself3, packed, norm, b3)

    out = pl.pallas_call(
        _final_body,
        grid=(nb,),
        in_specs=[
            pl.BlockSpec((F, TB), lambda i: (0, i)),
            pl.BlockSpec((1, TB), lambda i: (0, i)),
            pl.BlockSpec((F, 32), lambda i: (0, 0)),
            pl.BlockSpec((1, 32), lambda i: (0, 0)),
            pl.BlockSpec((32, 1), lambda i: (0, 0)),
            pl.BlockSpec((1, 1), lambda i: (0, 0)),
        ],
        out_specs=pl.BlockSpec((G, 1), lambda i: (0, 0)),
        out_shape=jax.ShapeDtypeStruct((G, 1), _f32),
        scratch_shapes=[
            pltpu.VMEM((F, G), _f32),
            pltpu.VMEM((1, G), _f32),
        ],
    )(h4t, jnp.reshape(batchP, (1, NP)), Wl,
      jnp.reshape(bl, (1, 32)), Wl2, jnp.reshape(bl2, (1, 1)))
    return out
